# Initial kernel scaffold; baseline (speedup 1.0000x reference)
import functools

import jax
import jax.numpy as jnp
from jax import lax
from jax.experimental import pallas as pl
from jax.experimental.pallas import tpu as pltpu
from jax.experimental.pallas import tpu_sc as plsc

N = 10000
E = 320000
D = 128

NC = 2
NS = 16
NW = NC * NS

EDGES_PER_W = E // NW
CHUNK = 125
NCHUNK = EDGES_PER_W // CHUNK
ROWS_PER_TILE = N // NS


def _sc_segment_partials(efeat, dst_idx3):
    mesh = plsc.VectorSubcoreMesh(core_axis_name="c", subcore_axis_name="s")

    @functools.partial(
        pl.kernel,
        out_type=jax.ShapeDtypeStruct((NC, N, D), jnp.float32),
        mesh=mesh,
        scratch_types=[
            pltpu.VMEM((NCHUNK, CHUNK), jnp.int32),
            pltpu.VMEM((CHUNK, D), jnp.float32),
            pltpu.VMEM((CHUNK, D), jnp.float32),
            pltpu.VMEM_SHARED((N, D), jnp.float32),
            pltpu.SemaphoreType.DMA,
            pltpu.SemaphoreType.DMA,
        ],
    )
    def k(efeat_hbm, idx_hbm, out_hbm, idx_v, ebuf0, ebuf1, acc, sem0, sem1):
        c = lax.axis_index("c")
        s = lax.axis_index("s")
        w = c * NS + s
        edge_base = w * EDGES_PER_W

        zeros16 = jnp.zeros((16,), jnp.float32)

        def zbody(i, _):
            for l in range(D // 16):
                ebuf0[i, pl.ds(l * 16, 16)] = zeros16
            return 0

        lax.fori_loop(0, CHUNK, zbody, 0)
        for t in range(ROWS_PER_TILE // CHUNK):
            pltpu.sync_copy(ebuf0, acc.at[pl.ds(s * ROWS_PER_TILE + t * CHUNK, CHUNK)])
        plsc.subcore_barrier()

        pltpu.sync_copy(idx_hbm.at[w], idx_v)

        def chunk_slice(j):
            return efeat_hbm.at[pl.ds(edge_base + j * CHUNK, CHUNK)]

        pltpu.async_copy(chunk_slice(0), ebuf0, sem0)
        pltpu.async_copy(chunk_slice(1), ebuf1, sem1)

        def body(it, _):
            j0 = it * 2
            pltpu.make_async_copy(chunk_slice(j0), ebuf0, sem0).wait()
            pltpu.sync_copy(ebuf0, acc.at[idx_v.at[j0]], add=True)

            @pl.when(j0 + 2 < NCHUNK)
            def _():
                pltpu.async_copy(chunk_slice(j0 + 2), ebuf0, sem0)

            pltpu.make_async_copy(chunk_slice(j0 + 1), ebuf1, sem1).wait()
            pltpu.sync_copy(ebuf1, acc.at[idx_v.at[j0 + 1]], add=True)

            @pl.when(j0 + 3 < NCHUNK)
            def _():
                pltpu.async_copy(chunk_slice(j0 + 3), ebuf1, sem1)

            return 0

        lax.fori_loop(0, NCHUNK // 2, body, 0)
        plsc.subcore_barrier()

        rows = pl.ds(s * ROWS_PER_TILE, ROWS_PER_TILE)
        pltpu.sync_copy(acc.at[rows], out_hbm.at[c, rows])

    return k(efeat, dst_idx3)


def _mlp_body(p_ref, nf_ref, w1a_ref, w1b_ref, b1_ref, w2_ref, b2_ref,
              sc_ref, bi_ref, out_ref):
    nf = nf_ref[...]
    agg = p_ref[0] + p_ref[1]
    x = (jnp.dot(agg, w1a_ref[...], preferred_element_type=jnp.float32)
         + jnp.dot(nf, w1b_ref[...], preferred_element_type=jnp.float32)
         + b1_ref[0])
    h = x * jax.nn.sigmoid(x)
    h = jnp.dot(h, w2_ref[...], preferred_element_type=jnp.float32) + b2_ref[0]
    mean = jnp.mean(h, axis=-1, keepdims=True)
    d = h - mean
    var = jnp.mean(d * d, axis=-1, keepdims=True)
    out_ref[...] = d * lax.rsqrt(var + 1e-5) * sc_ref[0] + bi_ref[0] + nf


def _tc_mlp(partials, nfeat, W1, b1, W2, b2, ln_scale, ln_bias):
    BLK = 1000
    grid = (N // BLK,)
    w1a = W1[:D]
    w1b = W1[D:]
    row2 = lambda a: a.reshape(1, D)
    return pl.pallas_call(
        _mlp_body,
        grid=grid,
        in_specs=[
            pl.BlockSpec((NC, BLK, D), lambda i: (0, i, 0)),
            pl.BlockSpec((BLK, D), lambda i: (i, 0)),
            pl.BlockSpec((D, D), lambda i: (0, 0)),
            pl.BlockSpec((D, D), lambda i: (0, 0)),
            pl.BlockSpec((1, D), lambda i: (0, 0)),
            pl.BlockSpec((D, D), lambda i: (0, 0)),
            pl.BlockSpec((1, D), lambda i: (0, 0)),
            pl.BlockSpec((1, D), lambda i: (0, 0)),
            pl.BlockSpec((1, D), lambda i: (0, 0)),
        ],
        out_specs=pl.BlockSpec((BLK, D), lambda i: (i, 0)),
        out_shape=jax.ShapeDtypeStruct((N, D), jnp.float32),
    )(partials, nfeat, w1a, w1b, row2(b1), W2, row2(b2),
      row2(ln_scale), row2(ln_bias))


@jax.jit
def kernel(efeat, nfeat, dst_idx, W1, b1, W2, b2, ln_scale, ln_bias):
    idx3 = dst_idx.astype(jnp.int32).reshape(NW, NCHUNK, CHUNK)
    partials = _sc_segment_partials(efeat, idx3)
    nfeat_new = _tc_mlp(partials, nfeat, W1, b1, W2, b2, ln_scale, ln_bias)
    return (efeat, nfeat_new)


# speedup vs baseline: 4.9622x; 4.9622x over previous
import functools

import jax
import jax.numpy as jnp
from jax import lax
from jax.experimental import pallas as pl
from jax.experimental.pallas import tpu as pltpu
from jax.experimental.pallas import tpu_sc as plsc

N = 10000
E = 320000
D = 128

NC = 2
NS = 16
NW = NC * NS

EDGES_PER_W = E // NW
CHUNK = 125
NCHUNK = EDGES_PER_W // CHUNK
ROWS_PER_TILE = N // NS


def _sc_segment_partials(efeat, dst_idx3):
    mesh = plsc.VectorSubcoreMesh(core_axis_name="c", subcore_axis_name="s")

    @functools.partial(
        pl.kernel,
        out_type=jax.ShapeDtypeStruct((NC, N, D), jnp.float32),
        mesh=mesh,
        compiler_params=pltpu.CompilerParams(use_tc_tiling_on_sc=False),
        scratch_types=[
            pltpu.VMEM((NCHUNK, CHUNK), jnp.int32),
            pltpu.VMEM((CHUNK, D), jnp.float32),
            pltpu.VMEM((CHUNK, D), jnp.float32),
            pltpu.VMEM_SHARED((N, D), jnp.float32),
            pltpu.SemaphoreType.DMA,
            pltpu.SemaphoreType.DMA,
        ],
    )
    def k(efeat_hbm, idx_hbm, out_hbm, idx_v, ebuf0, ebuf1, acc, sem0, sem1):
        c = lax.axis_index("c")
        s = lax.axis_index("s")
        w = c * NS + s
        edge_base = w * EDGES_PER_W

        zeros16 = jnp.zeros((16,), jnp.float32)

        def zbody(i, _):
            for l in range(D // 16):
                ebuf0[i, pl.ds(l * 16, 16)] = zeros16
            return 0

        lax.fori_loop(0, CHUNK, zbody, 0)
        for t in range(ROWS_PER_TILE // CHUNK):
            pltpu.sync_copy(ebuf0, acc.at[pl.ds(s * ROWS_PER_TILE + t * CHUNK, CHUNK)])
        plsc.subcore_barrier()

        pltpu.sync_copy(idx_hbm.at[w], idx_v)

        def chunk_slice(j):
            return efeat_hbm.at[pl.ds(edge_base + j * CHUNK, CHUNK)]

        pltpu.async_copy(chunk_slice(0), ebuf0, sem0)
        pltpu.async_copy(chunk_slice(1), ebuf1, sem1)

        def body(it, _):
            j0 = it * 2
            pltpu.make_async_copy(chunk_slice(j0), ebuf0, sem0).wait()
            pltpu.sync_copy(ebuf0, acc.at[idx_v.at[j0]], add=True)

            @pl.when(j0 + 2 < NCHUNK)
            def _():
                pltpu.async_copy(chunk_slice(j0 + 2), ebuf0, sem0)

            pltpu.make_async_copy(chunk_slice(j0 + 1), ebuf1, sem1).wait()
            pltpu.sync_copy(ebuf1, acc.at[idx_v.at[j0 + 1]], add=True)

            @pl.when(j0 + 3 < NCHUNK)
            def _():
                pltpu.async_copy(chunk_slice(j0 + 3), ebuf1, sem1)

            return 0

        lax.fori_loop(0, NCHUNK // 2, body, 0)
        plsc.subcore_barrier()

        rows = pl.ds(s * ROWS_PER_TILE, ROWS_PER_TILE)
        pltpu.sync_copy(acc.at[rows], out_hbm.at[c, rows])

    return k(efeat, dst_idx3)


def _mlp_body(p_ref, nf_ref, w1a_ref, w1b_ref, b1_ref, w2_ref, b2_ref,
              sc_ref, bi_ref, out_ref):
    nf = nf_ref[...]
    agg = p_ref[0] + p_ref[1]
    x = (jnp.dot(agg, w1a_ref[...], preferred_element_type=jnp.float32)
         + jnp.dot(nf, w1b_ref[...], preferred_element_type=jnp.float32)
         + b1_ref[0])
    h = x * jax.nn.sigmoid(x)
    h = jnp.dot(h, w2_ref[...], preferred_element_type=jnp.float32) + b2_ref[0]
    mean = jnp.mean(h, axis=-1, keepdims=True)
    d = h - mean
    var = jnp.mean(d * d, axis=-1, keepdims=True)
    out_ref[...] = d * lax.rsqrt(var + 1e-5) * sc_ref[0] + bi_ref[0] + nf


def _tc_mlp(partials, nfeat, W1, b1, W2, b2, ln_scale, ln_bias):
    BLK = 1000
    grid = (N // BLK,)
    w1a = W1[:D]
    w1b = W1[D:]
    row2 = lambda a: a.reshape(1, D)
    return pl.pallas_call(
        _mlp_body,
        grid=grid,
        in_specs=[
            pl.BlockSpec((NC, BLK, D), lambda i: (0, i, 0)),
            pl.BlockSpec((BLK, D), lambda i: (i, 0)),
            pl.BlockSpec((D, D), lambda i: (0, 0)),
            pl.BlockSpec((D, D), lambda i: (0, 0)),
            pl.BlockSpec((1, D), lambda i: (0, 0)),
            pl.BlockSpec((D, D), lambda i: (0, 0)),
            pl.BlockSpec((1, D), lambda i: (0, 0)),
            pl.BlockSpec((1, D), lambda i: (0, 0)),
            pl.BlockSpec((1, D), lambda i: (0, 0)),
        ],
        out_specs=pl.BlockSpec((BLK, D), lambda i: (i, 0)),
        out_shape=jax.ShapeDtypeStruct((N, D), jnp.float32),
    )(partials, nfeat, w1a, w1b, row2(b1), W2, row2(b2),
      row2(ln_scale), row2(ln_bias))


@jax.jit
def kernel(efeat, nfeat, dst_idx, W1, b1, W2, b2, ln_scale, ln_bias):
    idx3 = dst_idx.astype(jnp.int32).reshape(NW, NCHUNK, CHUNK)
    partials = _sc_segment_partials(efeat, idx3)
    nfeat_new = _tc_mlp(partials, nfeat, W1, b1, W2, b2, ln_scale, ln_bias)
    return (efeat, nfeat_new)


# speedup vs baseline: 4.9734x; 1.0023x over previous
import functools

import jax
import jax.numpy as jnp
from jax import lax
from jax.experimental import pallas as pl
from jax.experimental.pallas import tpu as pltpu
from jax.experimental.pallas import tpu_sc as plsc

N = 10000
E = 320000
D = 128

NC = 2
NS = 16
NW = NC * NS

EDGES_PER_W = E // NW
CHUNK = 125
NCHUNK = EDGES_PER_W // CHUNK
ROWS_PER_TILE = N // NS


def _sc_segment_partials(efeat, dst_idx3):
    mesh = plsc.VectorSubcoreMesh(core_axis_name="c", subcore_axis_name="s")

    @functools.partial(
        pl.kernel,
        out_type=jax.ShapeDtypeStruct((NC, N, D), jnp.float32),
        mesh=mesh,
        compiler_params=pltpu.CompilerParams(use_tc_tiling_on_sc=False),
        scratch_types=[
            pltpu.VMEM((NCHUNK, CHUNK), jnp.int32),
            pltpu.VMEM((CHUNK, D), jnp.float32),
            pltpu.VMEM((CHUNK, D), jnp.float32),
            pltpu.VMEM_SHARED((N, D), jnp.float32),
            pltpu.SemaphoreType.DMA,
            pltpu.SemaphoreType.DMA,
        ],
    )
    def k(efeat_hbm, idx_hbm, out_hbm, idx_v, ebuf0, ebuf1, acc, sem0, sem1):
        c = lax.axis_index("c")
        s = lax.axis_index("s")
        w = c * NS + s
        edge_base = w * EDGES_PER_W

        zeros16 = jnp.zeros((16,), jnp.float32)

        def zbody(i, _):
            for l in range(D // 16):
                ebuf0[i, pl.ds(l * 16, 16)] = zeros16
            return 0

        lax.fori_loop(0, CHUNK, zbody, 0)
        for t in range(ROWS_PER_TILE // CHUNK):
            pltpu.sync_copy(ebuf0, acc.at[pl.ds(s * ROWS_PER_TILE + t * CHUNK, CHUNK)])
        plsc.subcore_barrier()

        pltpu.sync_copy(idx_hbm.at[w], idx_v)

        def chunk_slice(j):
            return efeat_hbm.at[pl.ds(edge_base + j * CHUNK, CHUNK)]

        pltpu.async_copy(chunk_slice(0), ebuf0, sem0)
        pltpu.async_copy(chunk_slice(1), ebuf1, sem1)

        def body(it, _):
            j0 = it * 2
            pltpu.make_async_copy(chunk_slice(j0), ebuf0, sem0).wait()
            pltpu.sync_copy(ebuf0, acc.at[idx_v.at[j0]], add=True)

            @pl.when(j0 + 2 < NCHUNK)
            def _():
                pltpu.async_copy(chunk_slice(j0 + 2), ebuf0, sem0)

            pltpu.make_async_copy(chunk_slice(j0 + 1), ebuf1, sem1).wait()
            pltpu.sync_copy(ebuf1, acc.at[idx_v.at[j0 + 1]], add=True)

            @pl.when(j0 + 3 < NCHUNK)
            def _():
                pltpu.async_copy(chunk_slice(j0 + 3), ebuf1, sem1)

            return 0

        lax.fori_loop(0, NCHUNK // 2, body, 0)
        plsc.subcore_barrier()

        rows = pl.ds(s * ROWS_PER_TILE, ROWS_PER_TILE)
        pltpu.sync_copy(acc.at[rows], out_hbm.at[c, rows])

    return k(efeat, dst_idx3)


def _mlp_body(p_ref, nf_ref, w1a_ref, w1b_ref, b1_ref, w2_ref, b2_ref,
              sc_ref, bi_ref, out_ref):
    nf = nf_ref[...]
    agg = p_ref[0] + p_ref[1]
    x = (jnp.dot(agg, w1a_ref[...], preferred_element_type=jnp.float32)
         + jnp.dot(nf, w1b_ref[...], preferred_element_type=jnp.float32)
         + b1_ref[0])
    h = x * jax.nn.sigmoid(x)
    h = jnp.dot(h, w2_ref[...], preferred_element_type=jnp.float32) + b2_ref[0]
    mean = jnp.mean(h, axis=-1, keepdims=True)
    d = h - mean
    var = jnp.mean(d * d, axis=-1, keepdims=True)
    out_ref[...] = d * lax.rsqrt(var + 1e-5) * sc_ref[0] + bi_ref[0] + nf


def _tc_mlp(partials, nfeat, W1, b1, W2, b2, ln_scale, ln_bias):
    BLK = 1000
    grid = (N // BLK,)
    w1a = W1[:D]
    w1b = W1[D:]
    row2 = lambda a: a.reshape(1, D)
    return pl.pallas_call(
        _mlp_body,
        grid=grid,
        in_specs=[
            pl.BlockSpec((NC, BLK, D), lambda i: (0, i, 0)),
            pl.BlockSpec((BLK, D), lambda i: (i, 0)),
            pl.BlockSpec((D, D), lambda i: (0, 0)),
            pl.BlockSpec((D, D), lambda i: (0, 0)),
            pl.BlockSpec((1, D), lambda i: (0, 0)),
            pl.BlockSpec((D, D), lambda i: (0, 0)),
            pl.BlockSpec((1, D), lambda i: (0, 0)),
            pl.BlockSpec((1, D), lambda i: (0, 0)),
            pl.BlockSpec((1, D), lambda i: (0, 0)),
        ],
        out_specs=pl.BlockSpec((BLK, D), lambda i: (i, 0)),
        out_shape=jax.ShapeDtypeStruct((N, D), jnp.float32),
    )(partials, nfeat, w1a, w1b, row2(b1), W2, row2(b2),
      row2(ln_scale), row2(ln_bias))


@jax.jit
def _node_block(efeat, nfeat, dst_idx, W1, b1, W2, b2, ln_scale, ln_bias):
    idx3 = dst_idx.astype(jnp.int32).reshape(NW, NCHUNK, CHUNK)
    partials = _sc_segment_partials(efeat, idx3)
    return _tc_mlp(partials, nfeat, W1, b1, W2, b2, ln_scale, ln_bias)


def kernel(efeat, nfeat, dst_idx, W1, b1, W2, b2, ln_scale, ln_bias):
    return (efeat, _node_block(efeat, nfeat, dst_idx, W1, b1, W2, b2,
                               ln_scale, ln_bias))


# speedup vs baseline: 5.7787x; 1.1619x over previous
import functools

import jax
import jax.numpy as jnp
from jax import lax
from jax.experimental import pallas as pl
from jax.experimental.pallas import tpu as pltpu
from jax.experimental.pallas import tpu_sc as plsc

N = 10000
E = 320000
D = 128

NC = 2
NS = 16
NW = NC * NS

EDGES_PER_W = E // NW
CHUNK = 125
NCHUNK = EDGES_PER_W // CHUNK
NBUF = 2
ROWS_PER_TILE = N // NS


def _sc_segment_partials(efeat, dst_idx3):
    mesh = plsc.VectorSubcoreMesh(core_axis_name="c", subcore_axis_name="s")

    @functools.partial(
        pl.kernel,
        out_type=jax.ShapeDtypeStruct((NC, N, D), jnp.float32),
        mesh=mesh,
        compiler_params=pltpu.CompilerParams(use_tc_tiling_on_sc=False),
        scratch_types=[
            pltpu.VMEM((NCHUNK, CHUNK), jnp.int32),
            [pltpu.VMEM((CHUNK, D), jnp.float32) for _ in range(NBUF)],
            pltpu.VMEM_SHARED((N, D), jnp.float32),
            [pltpu.SemaphoreType.DMA for _ in range(NBUF)],
        ],
    )
    def k(efeat_hbm, idx_hbm, out_hbm, idx_v, ebufs, acc, sems):
        c = lax.axis_index("c")
        s = lax.axis_index("s")
        w = c * NS + s
        edge_base = w * EDGES_PER_W

        zeros16 = jnp.zeros((16,), jnp.float32)

        def zbody(i, _):
            for l in range(D // 16):
                ebufs[0][i, pl.ds(l * 16, 16)] = zeros16
            return 0

        lax.fori_loop(0, CHUNK, zbody, 0)
        for t in range(ROWS_PER_TILE // CHUNK):
            pltpu.sync_copy(ebufs[0], acc.at[pl.ds(s * ROWS_PER_TILE + t * CHUNK, CHUNK)])
        plsc.subcore_barrier()

        pltpu.sync_copy(idx_hbm.at[w], idx_v)

        def chunk_slice(j):
            return efeat_hbm.at[pl.ds(edge_base + j * CHUNK, CHUNK)]

        for b in range(NBUF):
            pltpu.async_copy(chunk_slice(b), ebufs[b], sems[b])

        def body(it, _):
            j0 = it * NBUF
            for b in range(NBUF):
                pltpu.make_async_copy(chunk_slice(j0 + b), ebufs[b], sems[b]).wait()
                pltpu.sync_copy(ebufs[b], acc.at[idx_v.at[j0 + b]], add=True)

                @pl.when(j0 + NBUF + b < NCHUNK)
                def _():
                    pltpu.async_copy(chunk_slice(j0 + NBUF + b), ebufs[b], sems[b])

            return 0

        lax.fori_loop(0, NCHUNK // NBUF, body, 0)
        plsc.subcore_barrier()

        rows = pl.ds(s * ROWS_PER_TILE, ROWS_PER_TILE)
        pltpu.sync_copy(acc.at[rows], out_hbm.at[c, rows])

    return k(efeat, dst_idx3)


def _mlp_body(p_ref, nf_ref, w1a_ref, w1b_ref, b1_ref, w2_ref, b2_ref,
              sc_ref, bi_ref, out_ref):
    nf = nf_ref[...]
    agg = p_ref[0] + p_ref[1]
    x = (jnp.dot(agg, w1a_ref[...], preferred_element_type=jnp.float32)
         + jnp.dot(nf, w1b_ref[...], preferred_element_type=jnp.float32)
         + b1_ref[0])
    h = x * jax.nn.sigmoid(x)
    h = jnp.dot(h, w2_ref[...], preferred_element_type=jnp.float32) + b2_ref[0]
    mean = jnp.mean(h, axis=-1, keepdims=True)
    d = h - mean
    var = jnp.mean(d * d, axis=-1, keepdims=True)
    out_ref[...] = d * lax.rsqrt(var + 1e-5) * sc_ref[0] + bi_ref[0] + nf


def _tc_mlp(partials, nfeat, W1, b1, W2, b2, ln_scale, ln_bias):
    BLK = 1000
    grid = (N // BLK,)
    w1a = W1[:D]
    w1b = W1[D:]
    row2 = lambda a: a.reshape(1, D)
    return pl.pallas_call(
        _mlp_body,
        grid=grid,
        in_specs=[
            pl.BlockSpec((NC, BLK, D), lambda i: (0, i, 0)),
            pl.BlockSpec((BLK, D), lambda i: (i, 0)),
            pl.BlockSpec((D, D), lambda i: (0, 0)),
            pl.BlockSpec((D, D), lambda i: (0, 0)),
            pl.BlockSpec((1, D), lambda i: (0, 0)),
            pl.BlockSpec((D, D), lambda i: (0, 0)),
            pl.BlockSpec((1, D), lambda i: (0, 0)),
            pl.BlockSpec((1, D), lambda i: (0, 0)),
            pl.BlockSpec((1, D), lambda i: (0, 0)),
        ],
        out_specs=pl.BlockSpec((BLK, D), lambda i: (i, 0)),
        out_shape=jax.ShapeDtypeStruct((N, D), jnp.float32),
    )(partials, nfeat, w1a, w1b, row2(b1), W2, row2(b2),
      row2(ln_scale), row2(ln_bias))


def _copy_body(in_ref, out_ref):
    out_ref[...] = in_ref[...]


def _tc_copy(efeat):
    BLK_E = 8000
    return pl.pallas_call(
        _copy_body,
        grid=(E // BLK_E,),
        in_specs=[pl.BlockSpec((BLK_E, D), lambda i: (i, 0))],
        out_specs=pl.BlockSpec((BLK_E, D), lambda i: (i, 0)),
        out_shape=jax.ShapeDtypeStruct((E, D), jnp.float32),
    )(efeat)


@jax.jit
def kernel(efeat, nfeat, dst_idx, W1, b1, W2, b2, ln_scale, ln_bias):
    idx3 = dst_idx.astype(jnp.int32).reshape(NW, NCHUNK, CHUNK)
    partials = _sc_segment_partials(efeat, idx3)
    efeat_out = _tc_copy(efeat)
    nfeat_new = _tc_mlp(partials, nfeat, W1, b1, W2, b2, ln_scale, ln_bias)
    return (efeat_out, nfeat_new)


# speedup vs baseline: 6.7494x; 1.1680x over previous
import functools

import jax
import jax.numpy as jnp
from jax import lax
from jax.experimental import pallas as pl
from jax.experimental.pallas import tpu as pltpu
from jax.experimental.pallas import tpu_sc as plsc

N = 10000
E = 320000
D = 128

NC = 2
NS = 16
NW = NC * NS

EDGES_PER_W = E // NW
CHUNK = 125
NCHUNK = EDGES_PER_W // CHUNK
NBUF = 2
ROWS_PER_TILE = N // NS


def _sc_segment_partials(efeat, dst_idx3):
    mesh = plsc.VectorSubcoreMesh(core_axis_name="c", subcore_axis_name="s")

    @functools.partial(
        pl.kernel,
        out_type=(jax.ShapeDtypeStruct((NC, N, D), jnp.float32),
                  jax.ShapeDtypeStruct((E, D), jnp.float32)),
        mesh=mesh,
        compiler_params=pltpu.CompilerParams(use_tc_tiling_on_sc=False),
        scratch_types=[
            pltpu.VMEM((NCHUNK, CHUNK), jnp.int32),
            [pltpu.VMEM((CHUNK, D), jnp.float32) for _ in range(NBUF)],
            pltpu.VMEM_SHARED((N, D), jnp.float32),
            [pltpu.SemaphoreType.DMA for _ in range(NBUF)],
            [pltpu.SemaphoreType.DMA for _ in range(NBUF)],
        ],
    )
    def k(efeat_hbm, idx_hbm, out_hbm, eout_hbm, idx_v, ebufs, acc, sems, wsems):
        c = lax.axis_index("c")
        s = lax.axis_index("s")
        w = c * NS + s
        edge_base = w * EDGES_PER_W

        zeros16 = jnp.zeros((16,), jnp.float32)

        def zbody(i, _):
            for l in range(D // 16):
                ebufs[0][i, pl.ds(l * 16, 16)] = zeros16
            return 0

        lax.fori_loop(0, CHUNK, zbody, 0)
        for t in range(ROWS_PER_TILE // CHUNK):
            pltpu.sync_copy(ebufs[0], acc.at[pl.ds(s * ROWS_PER_TILE + t * CHUNK, CHUNK)])
        plsc.subcore_barrier()

        pltpu.sync_copy(idx_hbm.at[w], idx_v)

        def chunk_slice(j):
            return efeat_hbm.at[pl.ds(edge_base + j * CHUNK, CHUNK)]

        def out_slice(j):
            return eout_hbm.at[pl.ds(edge_base + j * CHUNK, CHUNK)]

        for b in range(NBUF):
            pltpu.async_copy(chunk_slice(b), ebufs[b], sems[b])

        def body(it, _):
            j0 = it * NBUF
            for b in range(NBUF):
                pltpu.make_async_copy(chunk_slice(j0 + b), ebufs[b], sems[b]).wait()
                pltpu.sync_copy(ebufs[b], acc.at[idx_v.at[j0 + b]], add=True)
                pltpu.async_copy(ebufs[b], out_slice(j0 + b), wsems[b])

                @pl.when(j0 + NBUF + b < NCHUNK)
                def _():
                    pltpu.make_async_copy(ebufs[b], out_slice(j0 + b), wsems[b]).wait()
                    pltpu.async_copy(chunk_slice(j0 + NBUF + b), ebufs[b], sems[b])

            return 0

        lax.fori_loop(0, NCHUNK // NBUF, body, 0)
        for b in range(NBUF):
            pltpu.make_async_copy(ebufs[b], out_slice(NCHUNK - NBUF + b), wsems[b]).wait()
        plsc.subcore_barrier()

        rows = pl.ds(s * ROWS_PER_TILE, ROWS_PER_TILE)
        pltpu.sync_copy(acc.at[rows], out_hbm.at[c, rows])

    return k(efeat, dst_idx3)


def _mlp_body(p_ref, nf_ref, w1a_ref, w1b_ref, b1_ref, w2_ref, b2_ref,
              sc_ref, bi_ref, out_ref):
    nf = nf_ref[...]
    agg = p_ref[0] + p_ref[1]
    x = (jnp.dot(agg, w1a_ref[...], preferred_element_type=jnp.float32)
         + jnp.dot(nf, w1b_ref[...], preferred_element_type=jnp.float32)
         + b1_ref[0])
    h = x * jax.nn.sigmoid(x)
    h = jnp.dot(h, w2_ref[...], preferred_element_type=jnp.float32) + b2_ref[0]
    mean = jnp.mean(h, axis=-1, keepdims=True)
    d = h - mean
    var = jnp.mean(d * d, axis=-1, keepdims=True)
    out_ref[...] = d * lax.rsqrt(var + 1e-5) * sc_ref[0] + bi_ref[0] + nf


def _tc_mlp(partials, nfeat, W1, b1, W2, b2, ln_scale, ln_bias):
    BLK = 1000
    grid = (N // BLK,)
    w1a = W1[:D]
    w1b = W1[D:]
    row2 = lambda a: a.reshape(1, D)
    return pl.pallas_call(
        _mlp_body,
        grid=grid,
        in_specs=[
            pl.BlockSpec((NC, BLK, D), lambda i: (0, i, 0)),
            pl.BlockSpec((BLK, D), lambda i: (i, 0)),
            pl.BlockSpec((D, D), lambda i: (0, 0)),
            pl.BlockSpec((D, D), lambda i: (0, 0)),
            pl.BlockSpec((1, D), lambda i: (0, 0)),
            pl.BlockSpec((D, D), lambda i: (0, 0)),
            pl.BlockSpec((1, D), lambda i: (0, 0)),
            pl.BlockSpec((1, D), lambda i: (0, 0)),
            pl.BlockSpec((1, D), lambda i: (0, 0)),
        ],
        out_specs=pl.BlockSpec((BLK, D), lambda i: (i, 0)),
        out_shape=jax.ShapeDtypeStruct((N, D), jnp.float32),
    )(partials, nfeat, w1a, w1b, row2(b1), W2, row2(b2),
      row2(ln_scale), row2(ln_bias))


@jax.jit
def kernel(efeat, nfeat, dst_idx, W1, b1, W2, b2, ln_scale, ln_bias):
    idx3 = dst_idx.astype(jnp.int32).reshape(NW, NCHUNK, CHUNK)
    partials, efeat_out = _sc_segment_partials(efeat, idx3)
    nfeat_new = _tc_mlp(partials, nfeat, W1, b1, W2, b2, ln_scale, ln_bias)
    return (efeat_out, nfeat_new)


# speedup vs baseline: 7.1034x; 1.0524x over previous
import functools

import jax
import jax.numpy as jnp
from jax import lax
from jax.experimental import pallas as pl
from jax.experimental.pallas import tpu as pltpu
from jax.experimental.pallas import tpu_sc as plsc

N = 10000
E = 320000
D = 128

NC = 2
NS = 16
NW = NC * NS

EDGES_PER_W = E // NW
CHUNK = 125
NCHUNK = EDGES_PER_W // CHUNK
NBUF = 2
ROWS_PER_TILE = N // NS


def _sc_segment_partials(efeat, dst_idx3):
    mesh = plsc.VectorSubcoreMesh(core_axis_name="c", subcore_axis_name="s")

    @functools.partial(
        pl.kernel,
        out_type=(jax.ShapeDtypeStruct((NC, N, D), jnp.float32),
                  jax.ShapeDtypeStruct((E, D), jnp.float32)),
        mesh=mesh,
        compiler_params=pltpu.CompilerParams(use_tc_tiling_on_sc=False),
        scratch_types=[
            pltpu.VMEM((NCHUNK, CHUNK), jnp.int32),
            [pltpu.VMEM((CHUNK, D), jnp.float32) for _ in range(NBUF)],
            pltpu.VMEM_SHARED((N, D), jnp.float32),
            [pltpu.SemaphoreType.DMA for _ in range(NBUF)],
            [pltpu.SemaphoreType.DMA for _ in range(NBUF)],
        ],
    )
    def k(efeat_hbm, idx_hbm, out_hbm, eout_hbm, idx_v, ebufs, acc, sems, wsems):
        c = lax.axis_index("c")
        s = lax.axis_index("s")
        w = c * NS + s
        edge_base = w * EDGES_PER_W

        zeros16 = jnp.zeros((16,), jnp.float32)

        def zbody(i, _):
            for l in range(D // 16):
                ebufs[0][i, pl.ds(l * 16, 16)] = zeros16
            return 0

        lax.fori_loop(0, CHUNK, zbody, 0)
        for t in range(ROWS_PER_TILE // CHUNK):
            pltpu.sync_copy(ebufs[0], acc.at[pl.ds(s * ROWS_PER_TILE + t * CHUNK, CHUNK)])
        plsc.subcore_barrier()

        pltpu.sync_copy(idx_hbm.at[w], idx_v)

        def chunk_slice(j):
            return efeat_hbm.at[pl.ds(edge_base + j * CHUNK, CHUNK)]

        def out_slice(j):
            return eout_hbm.at[pl.ds(edge_base + j * CHUNK, CHUNK)]

        for b in range(NBUF):
            pltpu.async_copy(chunk_slice(b), ebufs[b], sems[b])

        def body(it, _):
            j0 = it * NBUF
            for b in range(NBUF):
                pltpu.make_async_copy(chunk_slice(j0 + b), ebufs[b], sems[b]).wait()
                pltpu.async_copy(ebufs[b], out_slice(j0 + b), wsems[b])
                pltpu.sync_copy(ebufs[b], acc.at[idx_v.at[j0 + b]], add=True)

                @pl.when(j0 + NBUF + b < NCHUNK)
                def _():
                    pltpu.make_async_copy(ebufs[b], out_slice(j0 + b), wsems[b]).wait()
                    pltpu.async_copy(chunk_slice(j0 + NBUF + b), ebufs[b], sems[b])

            return 0

        lax.fori_loop(0, NCHUNK // NBUF, body, 0)
        for b in range(NBUF):
            pltpu.make_async_copy(ebufs[b], out_slice(NCHUNK - NBUF + b), wsems[b]).wait()
        plsc.subcore_barrier()

        rows = pl.ds(s * ROWS_PER_TILE, ROWS_PER_TILE)
        pltpu.sync_copy(acc.at[rows], out_hbm.at[c, rows])

    return k(efeat, dst_idx3)


def _mlp_body(p_ref, nf_ref, w1a_ref, w1b_ref, b1_ref, w2_ref, b2_ref,
              sc_ref, bi_ref, out_ref):
    nf = nf_ref[...]
    agg = p_ref[0] + p_ref[1]
    x = (jnp.dot(agg, w1a_ref[...], preferred_element_type=jnp.float32)
         + jnp.dot(nf, w1b_ref[...], preferred_element_type=jnp.float32)
         + b1_ref[0])
    h = x * jax.nn.sigmoid(x)
    h = jnp.dot(h, w2_ref[...], preferred_element_type=jnp.float32) + b2_ref[0]
    mean = jnp.mean(h, axis=-1, keepdims=True)
    d = h - mean
    var = jnp.mean(d * d, axis=-1, keepdims=True)
    out_ref[...] = d * lax.rsqrt(var + 1e-5) * sc_ref[0] + bi_ref[0] + nf


def _tc_mlp(partials, nfeat, W1, b1, W2, b2, ln_scale, ln_bias):
    BLK = 1000
    grid = (N // BLK,)
    w1a = W1[:D]
    w1b = W1[D:]
    row2 = lambda a: a.reshape(1, D)
    return pl.pallas_call(
        _mlp_body,
        grid=grid,
        in_specs=[
            pl.BlockSpec((NC, BLK, D), lambda i: (0, i, 0)),
            pl.BlockSpec((BLK, D), lambda i: (i, 0)),
            pl.BlockSpec((D, D), lambda i: (0, 0)),
            pl.BlockSpec((D, D), lambda i: (0, 0)),
            pl.BlockSpec((1, D), lambda i: (0, 0)),
            pl.BlockSpec((D, D), lambda i: (0, 0)),
            pl.BlockSpec((1, D), lambda i: (0, 0)),
            pl.BlockSpec((1, D), lambda i: (0, 0)),
            pl.BlockSpec((1, D), lambda i: (0, 0)),
        ],
        out_specs=pl.BlockSpec((BLK, D), lambda i: (i, 0)),
        out_shape=jax.ShapeDtypeStruct((N, D), jnp.float32),
    )(partials, nfeat, w1a, w1b, row2(b1), W2, row2(b2),
      row2(ln_scale), row2(ln_bias))


@jax.jit
def kernel(efeat, nfeat, dst_idx, W1, b1, W2, b2, ln_scale, ln_bias):
    idx3 = dst_idx.astype(jnp.int32).reshape(NW, NCHUNK, CHUNK)
    partials, efeat_out = _sc_segment_partials(efeat, idx3)
    nfeat_new = _tc_mlp(partials, nfeat, W1, b1, W2, b2, ln_scale, ln_bias)
    return (efeat_out, nfeat_new)
